# baseline (speedup 1.0000x reference)
import functools

import jax
import jax.numpy as jnp
from jax import lax
from jax.experimental import pallas as pl
from jax.experimental.pallas import tpu as pltpu
from jax.experimental.pallas import tpu_sc as plsc

B = 16384
D = 64
V = 1000
NA = 5
NC = 2
NS = 16
NW = NC * NS
ROWS_PER_W = B // NW
CHUNK = 256
NCHUNKS = ROWS_PER_W // CHUNK
LANE = 16


def _project_body(cat_ref, col_ref, mat_ref, pat_ref, sea_ref, w_ref, b_ref,
                  o0, o1, o2, o3, o4):
    tabs = [cat_ref, col_ref, mat_ref, pat_ref, sea_ref]
    outs = [o0, o1, o2, o3, o4]
    for j in range(NA):
        r = jnp.dot(tabs[j][...], w_ref[j], preferred_element_type=jnp.float32)
        if j == 0:
            r = r + b_ref[...]
        outs[j][...] = r


def _project_tables(cat, col, mat, pat, sea, w3, b2):
    out_shape = [jax.ShapeDtypeStruct((V, D), jnp.float32) for _ in range(NA)]
    return pl.pallas_call(_project_body, out_shape=out_shape)(
        cat, col, mat, pat, sea, w3, b2)


_MESH = plsc.VectorSubcoreMesh(core_axis_name="c", subcore_axis_name="s",
                               num_cores=NC, num_subcores=NS)


@functools.partial(
    pl.kernel,
    out_type=jax.ShapeDtypeStruct((B, D), jnp.float32),
    mesh=_MESH,
    compiler_params=pltpu.CompilerParams(use_tc_tiling_on_sc=False),
    scratch_types=[
        [pltpu.VMEM((CHUNK,), jnp.int32) for _ in range(NA)],
        pltpu.VMEM((NA, CHUNK, D), jnp.float32),
        pltpu.VMEM((CHUNK, D), jnp.float32),
        pltpu.VMEM((D,), jnp.float32),
        pltpu.VMEM((D,), jnp.float32),
        pltpu.SemaphoreType.DMA,
    ],
)
def _sc_encode(idx_hbm, p0, p1, p2, p3, p4, gamma_hbm, beta_hbm, out_hbm,
               idx_v, gbuf, obuf, gam_v, bet_v, sem):
    tables = [p0, p1, p2, p3, p4]
    wid = lax.axis_index("s") * NC + lax.axis_index("c")

    pltpu.sync_copy(gamma_hbm, gam_v)
    pltpu.sync_copy(beta_hbm, bet_v)
    gam = [gam_v[pl.ds(v * LANE, LANE)] for v in range(D // LANE)]
    bet = [bet_v[pl.ds(v * LANE, LANE)] for v in range(D // LANE)]
    lane_iota = lax.iota(jnp.int32, LANE)
    perms = [jnp.bitwise_xor(lane_iota, k) for k in (8, 4, 2, 1)]

    def lane_sum(v):
        for p in perms:
            v = v + jnp.take_along_axis(v, p, axis=0, mode="promise_in_bounds")
        return v

    for c in range(NCHUNKS):
        base = wid * ROWS_PER_W + c * CHUNK
        for j in range(NA):
            pltpu.sync_copy(idx_hbm.at[pl.ds(j * B + base, CHUNK)], idx_v[j])
        copies = [
            pltpu.async_copy(tables[j].at[idx_v[j]], gbuf.at[j], sem)
            for j in range(NA)
        ]
        for cp in copies:
            cp.wait()

        @plsc.parallel_loop(0, CHUNK, 1, unroll=4)
        def _(r):
            acc = []
            for v in range(D // LANE):
                s = gbuf[0, r, pl.ds(v * LANE, LANE)]
                for j in range(1, NA):
                    s = s + gbuf[j, r, pl.ds(v * LANE, LANE)]
                acc.append(s)
            tot = (acc[0] + acc[1]) + (acc[2] + acc[3])
            sq = (acc[0] * acc[0] + acc[1] * acc[1]) + \
                 (acc[2] * acc[2] + acc[3] * acc[3])
            mean = lane_sum(tot) * (1.0 / D)
            var_s = lane_sum(sq)[0] * (1.0 / D)
            mean_s = mean[0]
            var_s = var_s - mean_s * mean_s + 1e-5
            bits = lax.bitcast_convert_type(var_s, jnp.int32)
            y = lax.bitcast_convert_type(
                jnp.int32(0x5F3759DF) - (bits >> 1), jnp.float32)
            for _ in range(3):
                y = y * (1.5 - 0.5 * var_s * y * y)
            rstd = jnp.broadcast_to(y, (LANE,))
            for v in range(D // LANE):
                o = (acc[v] - mean) * rstd * gam[v] + bet[v]
                obuf[r, pl.ds(v * LANE, LANE)] = jnp.maximum(o, 0.0)

        pltpu.sync_copy(obuf, out_hbm.at[pl.ds(base, CHUNK)])


def kernel(attributes, cat_table, col_table, mat_table, pat_table, sea_table,
           W, b, gamma, beta):
    idx = attributes.astype(jnp.int32).T.reshape(-1)
    w3 = W.reshape(NA, D, D)
    b2 = b.reshape(1, D)
    projected = _project_tables(cat_table, col_table, mat_table, pat_table,
                                sea_table, w3, b2)
    return _sc_encode(idx, *projected, gamma, beta)


# speedup vs baseline: 1.3390x; 1.3390x over previous
import functools

import jax
import jax.numpy as jnp
from jax import lax
from jax.experimental import pallas as pl
from jax.experimental.pallas import tpu as pltpu
from jax.experimental.pallas import tpu_sc as plsc

B = 16384
D = 64
V = 1000
NA = 5
NC = 2
NS = 16
NW = NC * NS
ROWS_PER_W = B // NW
CHUNK = 256
NCHUNKS = ROWS_PER_W // CHUNK
LANE = 16

_ROW_OFFS = (0, 1, 2000, 2001, 4000)


def _project_body(t0, t1, t2, t3, t4, wT_ref, b_ref, out_ref):
    def proj(tref, j):
        return lax.dot_general(tref[...], wT_ref[:, j * D:(j + 1) * D],
                               (((0,), (1,)), ((), ())),
                               preferred_element_type=jnp.float32)

    p0 = proj(t0, 0) + b_ref[...]
    p1 = proj(t1, 1)
    p2 = proj(t2, 2)
    p3 = proj(t3, 3)
    p4 = proj(t4, 4)
    out_ref[pl.ds(0, V), :] = jnp.concatenate([p0, p1], axis=1)
    out_ref[pl.ds(V, V), :] = jnp.concatenate([p2, p3], axis=1)
    out_ref[pl.ds(2 * V, V), :] = jnp.concatenate([p4, p4], axis=1)


def _project_tables(tabTs, wT, b2):
    return pl.pallas_call(
        _project_body,
        out_shape=jax.ShapeDtypeStruct((3 * V, 2 * D), jnp.float32),
    )(*tabTs, wT, b2)


@functools.cache
def _get_sc_encode():
    mesh = plsc.VectorSubcoreMesh(core_axis_name="c", subcore_axis_name="s",
                                  num_cores=NC, num_subcores=NS)
    return functools.partial(
        pl.kernel,
        out_type=jax.ShapeDtypeStruct((B, D), jnp.float32),
        mesh=mesh,
        compiler_params=pltpu.CompilerParams(use_tc_tiling_on_sc=False),
        scratch_types=[
            [pltpu.VMEM((CHUNK,), jnp.int32) for _ in range(NA)],
            pltpu.VMEM((NA, CHUNK, D), jnp.float32),
            pltpu.VMEM((CHUNK, D), jnp.float32),
            pltpu.VMEM((D,), jnp.float32),
            pltpu.VMEM((D,), jnp.float32),
            pltpu.SemaphoreType.DMA,
        ],
    )(_sc_encode_body)


def _sc_encode_body(idx_hbm, table_hbm, gamma_hbm, beta_hbm, out_hbm,
                    idx_v, gbuf, obuf, gam_v, bet_v, sem):
    wid = lax.axis_index("s") * NC + lax.axis_index("c")

    pltpu.sync_copy(gamma_hbm, gam_v)
    pltpu.sync_copy(beta_hbm, bet_v)
    gam = [gam_v[pl.ds(v * LANE, LANE)] for v in range(D // LANE)]
    bet = [bet_v[pl.ds(v * LANE, LANE)] for v in range(D // LANE)]
    lane_iota = lax.iota(jnp.int32, LANE)
    perms = [jnp.bitwise_xor(lane_iota, k) for k in (8, 4, 2, 1)]

    def lane_sum(v):
        for p in perms:
            v = v + jnp.take_along_axis(v, p, axis=0, mode="promise_in_bounds")
        return v

    for c in range(NCHUNKS):
        base = wid * ROWS_PER_W + c * CHUNK
        for j in range(NA):
            pltpu.sync_copy(idx_hbm.at[pl.ds(j * B + base, CHUNK)], idx_v[j])
        copies = [
            pltpu.async_copy(table_hbm.at[idx_v[j]], gbuf.at[j], sem)
            for j in range(NA)
        ]
        for cp in copies:
            cp.wait()

        @plsc.parallel_loop(0, CHUNK, 1, unroll=4)
        def _(r):
            acc = []
            for v in range(D // LANE):
                s = gbuf[0, r, pl.ds(v * LANE, LANE)]
                for j in range(1, NA):
                    s = s + gbuf[j, r, pl.ds(v * LANE, LANE)]
                acc.append(s)
            tot = (acc[0] + acc[1]) + (acc[2] + acc[3])
            sq = (acc[0] * acc[0] + acc[1] * acc[1]) + \
                 (acc[2] * acc[2] + acc[3] * acc[3])
            mean = lane_sum(tot) * (1.0 / D)
            var_s = lane_sum(sq)[0] * (1.0 / D)
            mean_s = mean[0]
            var_s = var_s - mean_s * mean_s + 1e-5
            bits = lax.bitcast_convert_type(var_s, jnp.int32)
            y = lax.bitcast_convert_type(
                jnp.int32(0x5F3759DF) - (bits >> 1), jnp.float32)
            for _ in range(3):
                y = y * (1.5 - 0.5 * var_s * y * y)
            rstd = jnp.broadcast_to(y, (LANE,))
            for v in range(D // LANE):
                o = (acc[v] - mean) * rstd * gam[v] + bet[v]
                obuf[r, pl.ds(v * LANE, LANE)] = jnp.maximum(o, 0.0)

        pltpu.sync_copy(obuf, out_hbm.at[pl.ds(base, CHUNK)])


def kernel(attributes, cat_table, col_table, mat_table, pat_table, sea_table,
           W, b, gamma, beta):
    at = attributes.astype(jnp.int32)
    offs = jnp.array(_ROW_OFFS, jnp.int32)
    idx = (at * 2 + offs[None, :]).T.reshape(-1)
    tabTs = [t.T for t in (cat_table, col_table, mat_table, pat_table,
                           sea_table)]
    p_all = _project_tables(tabTs, W.T, b.reshape(1, D))
    table = p_all.reshape(6 * V, D)
    return _get_sc_encode()(idx, table, gamma, beta)
